# baseline (speedup 1.0000x reference)
import functools

import jax
import jax.numpy as jnp
from jax import lax
from jax.experimental import pallas as pl
from jax.experimental.pallas import tpu as pltpu
from jax.experimental.pallas import tpu_sc as plsc

_D = 64
_GSZ = 128
_NW = 32
_G = 2


@functools.cache
def _build(n_groups: int, interpret: bool = False):
    per_w = n_groups // _NW
    n_chunks = per_w // _G
    assert n_chunks % 2 == 0 and n_chunks >= 4
    mesh = plsc.VectorSubcoreMesh(core_axis_name="c", subcore_axis_name="s")

    @functools.partial(
        pl.kernel,
        out_type=jax.ShapeDtypeStruct((n_groups, _GSZ, _D), jnp.float32),
        mesh=mesh,
        scratch_types=[
            pltpu.VMEM((per_w, _GSZ), jnp.int32),
            pltpu.VMEM((per_w, _GSZ), jnp.int32),
            pltpu.VMEM((2, _G, _GSZ, _D), jnp.float32),
            pltpu.VMEM((2, _G, _GSZ, _D), jnp.float32),
            pltpu.SemaphoreType.DMA,
            pltpu.SemaphoreType.DMA,
            pltpu.SemaphoreType.DMA,
            pltpu.SemaphoreType.DMA,
        ],
        compiler_params=pltpu.CompilerParams(use_tc_tiling_on_sc=False),
        interpret=interpret,
    )
    def k(row_idx, col_idx, row_tab, col_tab, out,
          ridx, cidx, rows, cols, sg0, sg1, so0, so1):
        wid = lax.axis_index("s") * 2 + lax.axis_index("c")
        base = wid * per_w
        sg = (sg0, sg1)
        so = (so0, so1)

        pltpu.sync_copy(row_idx.at[pl.ds(base, per_w)], ridx)
        pltpu.sync_copy(col_idx.at[pl.ds(base, per_w)], cidx)

        def fire(ci, b):
            g0 = ci * _G
            for j in range(_G):
                pltpu.async_copy(row_tab.at[ridx.at[g0 + j]], rows.at[b, j], sg[b])
                pltpu.async_copy(col_tab.at[cidx.at[g0 + j]], cols.at[b, j], sg[b])

        def wait_gathers(ci, b):
            g0 = ci * _G
            for j in range(_G):
                pltpu.make_async_copy(
                    row_tab.at[ridx.at[g0 + j]], rows.at[b, j], sg[b]).wait()
                pltpu.make_async_copy(
                    col_tab.at[cidx.at[g0 + j]], cols.at[b, j], sg[b]).wait()

        def fire_out(ci, b):
            pltpu.async_copy(rows.at[b], out.at[pl.ds(base + ci * _G, _G)], so[b])

        def wait_out(ci, b):
            pltpu.make_async_copy(
                rows.at[b], out.at[pl.ds(base + ci * _G, _G)], so[b]).wait()

        def add_chunk(b):
            for j in range(_G):
                @pl.loop(0, _GSZ, unroll=4)
                def _el(e):
                    for kk in range(_D // 16):
                        sl = pl.ds(kk * 16, 16)
                        rows[b, j, e, sl] = rows[b, j, e, sl] + cols[b, j, e, sl]

        fire(0, 0)
        fire(1, 1)
        wait_gathers(0, 0)
        add_chunk(0)
        fire_out(0, 0)

        @pl.loop(0, (n_chunks - 2) // 2)
        def _pair(p):
            for b, off in ((1, 1), (0, 2)):
                ci = p * 2 + off
                wait_out(ci - 1, 1 - b)
                fire(ci + 1, 1 - b)
                wait_gathers(ci, b)
                add_chunk(b)
                fire_out(ci, b)

        wait_out(n_chunks - 2, 0)
        wait_gathers(n_chunks - 1, 1)
        add_chunk(1)
        fire_out(n_chunks - 1, 1)
        wait_out(n_chunks - 1, 1)

    return k


def kernel(row_indices, col_indices, row_table, col_table):
    b, l = row_indices.shape
    n = b * l
    n_groups = n // _GSZ
    ri = row_indices.reshape(n_groups, _GSZ).astype(jnp.int32)
    ci = col_indices.reshape(n_groups, _GSZ).astype(jnp.int32)
    out = _build(n_groups)(ri, ci, row_table, col_table)
    return out.reshape(b, l, _D)
